# baseline (speedup 1.0000x reference)
import math

import jax
import jax.numpy as jnp
from jax import lax
from jax.experimental import pallas as pl
from jax.experimental.pallas import tpu as pltpu

_NS = 2
_X_CHUNK = 1000
_BM = 112


def _prelu(h, alpha):
    return jnp.where(h >= 0.0, h, alpha * h)


def _body(x_ref, w1_ref, w2_ref, b1_ref, a1_ref, b2_ref, a2_ref,
          adjr0, adjr1, o_ref, y_scr, h1_scr):
    n = y_scr.shape[0]
    nf = n // _X_CHUNK
    rps = n // _NS
    na = math.ceil(rps / _BM)
    i = pl.program_id(0)

    @pl.when(i < nf)
    def _feat1():
        row = pl.multiple_of(i * _X_CHUNK, _X_CHUNK)
        y_scr[pl.ds(row, _X_CHUNK), :] = lax.dot_general(
            x_ref[...], w1_ref[...], (((1,), (1,)), ((), ())),
            preferred_element_type=jnp.float32)

    @pl.when(jnp.logical_and(i >= nf, i < nf + na))
    def _agg1():
        j = i - nf
        row = pl.multiple_of(j * _BM, _BM)
        alpha = a1_ref[0, 0]
        for q, a_ref in enumerate((adjr0, adjr1)):
            acc = lax.dot_general(
                a_ref[0], y_scr[...], (((1,), (0,)), ((), ())),
                preferred_element_type=jnp.float32)
            h1_scr[q, pl.ds(row, _BM), :] = _prelu(
                acc + b1_ref[...], alpha).astype(h1_scr.dtype)

    @pl.when(i == nf + na)
    def _feat2():
        for q in range(_NS):
            for c in range(rps // _X_CHUNK):
                dst = q * rps + c * _X_CHUNK
                y_scr[pl.ds(dst, _X_CHUNK), :] = lax.dot_general(
                    h1_scr[q, pl.ds(c * _X_CHUNK, _X_CHUNK), :], w2_ref[...],
                    (((1,), (1,)), ((), ())),
                    preferred_element_type=jnp.float32)

    @pl.when(i > nf + na)
    def _agg2():
        alpha = a2_ref[0, 0]
        for q, a_ref in enumerate((adjr0, adjr1)):
            acc = lax.dot_general(
                a_ref[0], y_scr[...], (((1,), (0,)), ((), ())),
                preferred_element_type=jnp.float32)
            o_ref[q] = _prelu(acc + b2_ref[...], alpha)


def kernel(features, seq1, adj, b1, W1, a1, b2, W2, a2, sparse):
    del seq1, sparse
    x = features[0]
    n, d_in = x.shape
    d_out = W1.shape[0]
    adj3 = adj[0].reshape(_NS, n // _NS, n)
    nf = n // _X_CHUNK
    rps = n // _NS
    na = math.ceil(rps / _BM)
    grid = (nf + na + 1 + na,)

    def _x_map(i):
        return (jnp.minimum(i, nf - 1), 0)

    def _adj_map_for(q):
        def _m(i):
            j = jnp.where(i < nf + na, jnp.clip(i - nf, 0, na - 1),
                          jnp.clip(i - (nf + na + 1), 0, na - 1))
            return (q, j, 0)
        return _m

    def _out_map(i):
        return (0, jnp.clip(i - (nf + na + 1), 0, na - 1), 0)

    _const = lambda i: (0, 0)
    h2 = pl.pallas_call(
        _body,
        grid=grid,
        in_specs=[
            pl.BlockSpec((_X_CHUNK, d_in), _x_map),
            pl.BlockSpec((d_out, d_in), _const),
            pl.BlockSpec((d_out, d_out), _const),
            pl.BlockSpec((1, d_out), _const),
            pl.BlockSpec((1, 1), _const),
            pl.BlockSpec((1, d_out), _const),
            pl.BlockSpec((1, 1), _const),
        ] + [
            pl.BlockSpec((1, _BM, n), _adj_map_for(q)) for q in range(_NS)
        ],
        out_specs=pl.BlockSpec((_NS, _BM, d_out), _out_map),
        out_shape=jax.ShapeDtypeStruct((_NS, rps, d_out), jnp.float32),
        scratch_shapes=[
            pltpu.VMEM((n, d_out), jnp.float32),
            pltpu.VMEM((_NS, na * _BM, d_out), jnp.bfloat16),
        ],
    )(x, W1, W2.astype(jnp.bfloat16), b1.reshape(1, -1), a1.reshape(1, 1),
      b2.reshape(1, -1), a2.reshape(1, 1), *([adj3] * _NS))
    return h2.reshape(n, d_out)[None]


# speedup vs baseline: 1.0909x; 1.0909x over previous
import math

import jax
import jax.numpy as jnp
from jax import lax
from jax.experimental import pallas as pl
from jax.experimental.pallas import tpu as pltpu

_NS = 2
_X_CHUNK = 400
_BM = 200


def _prelu(h, alpha):
    return jnp.where(h >= 0.0, h, alpha * h)


def _body(x_ref, w1_ref, w2_ref, b1_ref, a1_ref, b2_ref, a2_ref,
          adjr0, adjr1, o_ref, y_scr, h1_scr):
    n = y_scr.shape[0]
    nf = n // _X_CHUNK
    rps = n // _NS
    na = math.ceil(rps / _BM)
    i = pl.program_id(0)

    @pl.when(i < nf)
    def _feat1():
        row = pl.multiple_of(i * _X_CHUNK, _X_CHUNK)
        y_scr[pl.ds(row, _X_CHUNK), :] = lax.dot_general(
            x_ref[...], w1_ref[...], (((1,), (1,)), ((), ())),
            preferred_element_type=jnp.float32).astype(y_scr.dtype)

    @pl.when(jnp.logical_and(i >= nf, i < nf + na))
    def _agg1():
        j = i - nf
        row = pl.multiple_of(j * _BM, _BM)
        alpha = a1_ref[0, 0]
        for q, a_ref in enumerate((adjr0, adjr1)):
            acc = lax.dot_general(
                a_ref[0].astype(jnp.bfloat16), y_scr[...],
                (((1,), (0,)), ((), ())),
                preferred_element_type=jnp.float32)
            h1_scr[q, pl.ds(row, _BM), :] = _prelu(
                acc + b1_ref[...], alpha).astype(h1_scr.dtype)

    @pl.when(i == nf + na)
    def _feat2():
        ck = 1000
        for q in range(_NS):
            for c in range(rps // ck):
                dst = q * rps + c * ck
                y_scr[pl.ds(dst, ck), :] = lax.dot_general(
                    h1_scr[q, pl.ds(c * ck, ck), :], w2_ref[...],
                    (((1,), (1,)), ((), ())),
                    preferred_element_type=jnp.float32).astype(y_scr.dtype)

    @pl.when(i > nf + na)
    def _agg2():
        alpha = a2_ref[0, 0]
        for q, a_ref in enumerate((adjr0, adjr1)):
            acc = lax.dot_general(
                a_ref[0].astype(jnp.bfloat16), y_scr[...],
                (((1,), (0,)), ((), ())),
                preferred_element_type=jnp.float32)
            o_ref[q] = _prelu(acc + b2_ref[...], alpha)


def kernel(features, seq1, adj, b1, W1, a1, b2, W2, a2, sparse):
    del seq1, sparse
    x = features[0]
    n, d_in = x.shape
    d_out = W1.shape[0]
    adj3 = adj[0].reshape(_NS, n // _NS, n)
    nf = n // _X_CHUNK
    rps = n // _NS
    na = math.ceil(rps / _BM)
    grid = (nf + na + 1 + na,)

    def _x_map(i):
        return (jnp.minimum(i, nf - 1), 0)

    def _adj_map_for(q):
        def _m(i):
            j = jnp.where(i < nf + na, jnp.clip(i - nf, 0, na - 1),
                          jnp.clip(i - (nf + na + 1), 0, na - 1))
            return (q, j, 0)
        return _m

    def _out_map(i):
        return (0, jnp.clip(i - (nf + na + 1), 0, na - 1), 0)

    _const = lambda i: (0, 0)
    h2 = pl.pallas_call(
        _body,
        grid=grid,
        in_specs=[
            pl.BlockSpec((_X_CHUNK, d_in), _x_map),
            pl.BlockSpec((d_out, d_in), _const),
            pl.BlockSpec((d_out, d_out), _const),
            pl.BlockSpec((1, d_out), _const),
            pl.BlockSpec((1, 1), _const),
            pl.BlockSpec((1, d_out), _const),
            pl.BlockSpec((1, 1), _const),
        ] + [
            pl.BlockSpec((1, _BM, n), _adj_map_for(q)) for q in range(_NS)
        ],
        out_specs=pl.BlockSpec((_NS, _BM, d_out), _out_map),
        out_shape=jax.ShapeDtypeStruct((_NS, rps, d_out), jnp.float32),
        scratch_shapes=[
            pltpu.VMEM((n, d_out), jnp.bfloat16),
            pltpu.VMEM((_NS, na * _BM, d_out), jnp.bfloat16),
        ],
    )(x, W1, W2.astype(jnp.bfloat16), b1.reshape(1, -1), a1.reshape(1, 1),
      b2.reshape(1, -1), a2.reshape(1, 1), *([adj3] * _NS))
    return h2.reshape(n, d_out)[None]


# speedup vs baseline: 1.0988x; 1.0073x over previous
import math

import jax
import jax.numpy as jnp
from jax import lax
from jax.experimental import pallas as pl
from jax.experimental.pallas import tpu as pltpu

_NS = 2
_X_CHUNK = 400
_BM = 200


def _prelu(h, alpha):
    return jnp.where(h >= 0.0, h, alpha * h)


def _body(x_ref, w1_ref, w2_ref, b1_ref, a1_ref, b2_ref, a2_ref,
          adjr0, adjr1, o_ref, y_scr, h1_scr):
    n = y_scr.shape[0]
    nf = n // _X_CHUNK
    rps = n // _NS
    na = math.ceil(rps / _BM)
    i = pl.program_id(0)

    @pl.when(i < nf)
    def _feat1():
        row = pl.multiple_of(i * _X_CHUNK, _X_CHUNK)
        y_scr[pl.ds(row, _X_CHUNK), :] = lax.dot_general(
            x_ref[...], w1_ref[...], (((1,), (1,)), ((), ())),
            preferred_element_type=jnp.float32).astype(y_scr.dtype)

    @pl.when(jnp.logical_and(i >= nf, i < nf + na))
    def _agg1():
        j = i - nf
        row = pl.multiple_of(j * _BM, _BM)
        alpha = a1_ref[0, 0]
        for q, a_ref in enumerate((adjr0, adjr1)):
            acc = lax.dot_general(
                a_ref[0], y_scr[...], (((1,), (0,)), ((), ())),
                preferred_element_type=jnp.float32)
            h1_scr[q, pl.ds(row, _BM), :] = _prelu(
                acc + b1_ref[...], alpha).astype(h1_scr.dtype)

    @pl.when(i == nf + na)
    def _feat2():
        ck = 1000
        for q in range(_NS):
            for c in range(rps // ck):
                dst = q * rps + c * ck
                y_scr[pl.ds(dst, ck), :] = lax.dot_general(
                    h1_scr[q, pl.ds(c * ck, ck), :], w2_ref[...],
                    (((1,), (1,)), ((), ())),
                    preferred_element_type=jnp.float32).astype(y_scr.dtype)

    @pl.when(i > nf + na)
    def _agg2():
        alpha = a2_ref[0, 0]
        for q, a_ref in enumerate((adjr0, adjr1)):
            acc = lax.dot_general(
                a_ref[0], y_scr[...], (((1,), (0,)), ((), ())),
                preferred_element_type=jnp.float32)
            o_ref[q] = _prelu(acc + b2_ref[...], alpha)


def kernel(features, seq1, adj, b1, W1, a1, b2, W2, a2, sparse):
    del seq1, sparse
    x = features[0]
    n, d_in = x.shape
    d_out = W1.shape[0]
    adj3 = adj[0].reshape(_NS, n // _NS, n)
    nf = n // _X_CHUNK
    rps = n // _NS
    na = math.ceil(rps / _BM)
    grid = (nf + na + 1 + na,)

    def _x_map(i):
        return (jnp.minimum(i, nf - 1), 0)

    def _adj_map_for(q):
        def _m(i):
            j = jnp.where(i < nf + na, jnp.clip(i - nf, 0, na - 1),
                          jnp.clip(i - (nf + na + 1), 0, na - 1))
            return (q, j, 0)
        return _m

    def _out_map(i):
        return (0, jnp.clip(i - (nf + na + 1), 0, na - 1), 0)

    _const = lambda i: (0, 0)
    h2 = pl.pallas_call(
        _body,
        grid=grid,
        in_specs=[
            pl.BlockSpec((_X_CHUNK, d_in), _x_map),
            pl.BlockSpec((d_out, d_in), _const),
            pl.BlockSpec((d_out, d_out), _const),
            pl.BlockSpec((1, d_out), _const),
            pl.BlockSpec((1, 1), _const),
            pl.BlockSpec((1, d_out), _const),
            pl.BlockSpec((1, 1), _const),
        ] + [
            pl.BlockSpec((1, _BM, n), _adj_map_for(q)) for q in range(_NS)
        ],
        out_specs=pl.BlockSpec((_NS, _BM, d_out), _out_map),
        out_shape=jax.ShapeDtypeStruct((_NS, rps, d_out), jnp.float32),
        scratch_shapes=[
            pltpu.VMEM((n, d_out), jnp.bfloat16),
            pltpu.VMEM((_NS, na * _BM, d_out), jnp.bfloat16),
        ],
    )(x, W1, W2.astype(jnp.bfloat16), b1.reshape(1, -1), a1.reshape(1, 1),
      b2.reshape(1, -1), a2.reshape(1, 1), *([adj3] * _NS))
    return h2.reshape(n, d_out)[None]


# speedup vs baseline: 1.1074x; 1.0078x over previous
import jax
import jax.numpy as jnp
from jax import lax
from jax.experimental import pallas as pl
from jax.experimental.pallas import tpu as pltpu

_NS = 2
_X_CHUNK = 400
_BM = 200
_Y2_CHUNK = 1000


def _prelu(h, alpha):
    return jnp.where(h >= 0.0, h, alpha * h)


def _body(x_ref, w1_ref, w2_ref, b1_ref, a1_ref, b2_ref, a2_ref,
          adjr0, adjr1, o_ref, y_scr, h1_scr):
    n = y_scr.shape[0]
    nf = n // _X_CHUNK
    rps = n // _NS
    na = rps // _BM
    steps_per_chunk = _Y2_CHUNK // _BM
    n_chunks = rps // _Y2_CHUNK
    i = pl.program_id(0)

    def _y2_chunk(c):
        for q in range(_NS):
            sl = pl.ds(c * _Y2_CHUNK, _Y2_CHUNK)
            h1_scr[q, sl, :] = lax.dot_general(
                h1_scr[q, sl, :], w2_ref[...], (((1,), (1,)), ((), ())),
                preferred_element_type=jnp.float32).astype(h1_scr.dtype)

    @pl.when(i < nf)
    def _feat1():
        row = pl.multiple_of(i * _X_CHUNK, _X_CHUNK)
        y_scr[pl.ds(row, _X_CHUNK), :] = lax.dot_general(
            x_ref[...], w1_ref[...], (((1,), (1,)), ((), ())),
            preferred_element_type=jnp.float32).astype(y_scr.dtype)

    @pl.when(jnp.logical_and(i >= nf, i < nf + na))
    def _agg1():
        j = i - nf
        row = pl.multiple_of(j * _BM, _BM)
        alpha = a1_ref[0, 0]
        for q, a_ref in enumerate((adjr0, adjr1)):
            acc = lax.dot_general(
                a_ref[0], y_scr[...], (((1,), (0,)), ((), ())),
                preferred_element_type=jnp.float32)
            h1_scr[q, pl.ds(row, _BM), :] = _prelu(
                acc + b1_ref[...], alpha).astype(h1_scr.dtype)
        for c in range(n_chunks - 1):
            @pl.when(j == (c + 1) * steps_per_chunk)
            def _(c=c):
                _y2_chunk(c)

    @pl.when(i >= nf + na)
    def _agg2():
        j = i - (nf + na)

        @pl.when(j == 0)
        def _():
            _y2_chunk(n_chunks - 1)

        alpha = a2_ref[0, 0]
        for q, a_ref in enumerate((adjr0, adjr1)):
            acc = lax.dot_general(
                a_ref[0][:, :rps], h1_scr[0], (((1,), (0,)), ((), ())),
                preferred_element_type=jnp.float32)
            acc = acc + lax.dot_general(
                a_ref[0][:, rps:], h1_scr[1], (((1,), (0,)), ((), ())),
                preferred_element_type=jnp.float32)
            o_ref[q] = _prelu(acc + b2_ref[...], alpha)


def kernel(features, seq1, adj, b1, W1, a1, b2, W2, a2, sparse):
    del seq1, sparse
    x = features[0]
    n, d_in = x.shape
    d_out = W1.shape[0]
    adj3 = adj[0].reshape(_NS, n // _NS, n)
    nf = n // _X_CHUNK
    rps = n // _NS
    na = rps // _BM
    grid = (nf + 2 * na,)

    def _x_map(i):
        return (jnp.minimum(i, nf - 1), 0)

    def _adj_map_for(q):
        def _m(i):
            j = jnp.where(i < nf + na, jnp.clip(i - nf, 0, na - 1),
                          jnp.clip(i - (nf + na), 0, na - 1))
            return (q, j, 0)
        return _m

    def _out_map(i):
        return (0, jnp.clip(i - (nf + na), 0, na - 1), 0)

    _const = lambda i: (0, 0)
    h2 = pl.pallas_call(
        _body,
        grid=grid,
        in_specs=[
            pl.BlockSpec((_X_CHUNK, d_in), _x_map),
            pl.BlockSpec((d_out, d_in), _const),
            pl.BlockSpec((d_out, d_out), _const),
            pl.BlockSpec((1, d_out), _const),
            pl.BlockSpec((1, 1), _const),
            pl.BlockSpec((1, d_out), _const),
            pl.BlockSpec((1, 1), _const),
        ] + [
            pl.BlockSpec((1, _BM, n), _adj_map_for(q)) for q in range(_NS)
        ],
        out_specs=pl.BlockSpec((_NS, _BM, d_out), _out_map),
        out_shape=jax.ShapeDtypeStruct((_NS, rps, d_out), jnp.float32),
        scratch_shapes=[
            pltpu.VMEM((n, d_out), jnp.bfloat16),
            pltpu.VMEM((_NS, rps, d_out), jnp.bfloat16),
        ],
    )(x, W1, W2, b1.reshape(1, -1), a1.reshape(1, 1),
      b2.reshape(1, -1), a2.reshape(1, 1), *([adj3] * _NS))
    return h2.reshape(n, d_out)[None]
